# Initial kernel scaffold; baseline (speedup 1.0000x reference)
import jax
import jax.numpy as jnp
from jax.experimental import pallas as pl


def kernel(adj, input, W, b_lin, bias):
    raise NotImplementedError("write your pallas kernel here")


# speedup vs baseline: 1.0289x; 1.0289x over previous
import jax
import jax.numpy as jnp
from jax.experimental import pallas as pl
from jax.experimental.pallas import tpu as pltpu

_BM = 400


def _gcn_kernel(adj_ref, x_ref, wt_ref, bl_ref, bias_ref, out_ref, support_ref):
    @pl.when(pl.program_id(0) == 0)
    def _compute_support():
        support_ref[...] = (
            jnp.dot(x_ref[...], wt_ref[...], preferred_element_type=jnp.float32)
            + bl_ref[...]
        )

    out_ref[...] = (
        jnp.dot(adj_ref[...], support_ref[...], preferred_element_type=jnp.float32)
        + bias_ref[...]
    )


@jax.jit
def kernel(adj, input, W, b_lin, bias):
    n, d_in = input.shape
    d_out = W.shape[0]
    wt = W.T
    bl = b_lin.reshape(1, d_out)
    grid = (n // _BM,)
    return pl.pallas_call(
        _gcn_kernel,
        grid=grid,
        in_specs=[
            pl.BlockSpec((_BM, n), lambda i: (i, 0)),
            pl.BlockSpec((n, d_in), lambda i: (0, 0)),
            pl.BlockSpec((d_in, d_out), lambda i: (0, 0)),
            pl.BlockSpec((1, d_out), lambda i: (0, 0)),
            pl.BlockSpec((1, d_out), lambda i: (0, 0)),
        ],
        out_specs=pl.BlockSpec((_BM, d_out), lambda i: (i, 0)),
        out_shape=jax.ShapeDtypeStruct((n, d_out), jnp.float32),
        scratch_shapes=[pltpu.VMEM((n, d_out), jnp.float32)],
    )(adj, input, wt, bl, bias)
